# Initial kernel scaffold; baseline (speedup 1.0000x reference)
import jax
import jax.numpy as jnp
from jax.experimental import pallas as pl


def kernel(anchors, objectness, box_regression):
    raise NotImplementedError("write your pallas kernel here")


# speedup vs baseline: 6.4111x; 6.4111x over previous
import jax
import jax.numpy as jnp
import numpy as np
from jax.experimental import pallas as pl
from jax.experimental.pallas import tpu as pltpu

_PRE = 1000
_POST = 300
_NMS_THRESH = 0.7
_IM = 800.0
_CLIP = float(np.log(1000.0 / 16.0))
_NEG = -1e9
_CHUNK = 2400


def _dotT(a, b):
    return jax.lax.dot_general(
        a, b, (((0,), (0,)), ((), ())), preferred_element_type=jnp.float32
    )


def _rpn_kernel(data_ref, idx_ref, obj_ref, out_ref, iou_s):
    data = data_ref[0]
    idx = idx_ref[0]
    logits = obj_ref[0]

    num = data.shape[0]
    gathered = jnp.zeros((_PRE, 8), jnp.float32)
    for c in range(0, num, _CHUNK):
        blk = data[c:c + _CHUNK, :]
        cols = c + jax.lax.broadcasted_iota(jnp.int32, (_PRE, _CHUNK), 1)
        oh = (idx == cols).astype(jnp.float32)
        gathered = gathered + jnp.dot(
            oh, blk, preferred_element_type=jnp.float32
        )

    anc = gathered[:, 0:4]
    reg = gathered[:, 4:8]
    scores = jax.nn.sigmoid(logits)

    lows, highs = [], []
    for d in range(2):
        size = anc[:, d + 2:d + 3] - anc[:, d:d + 1]
        ctr = anc[:, d:d + 1] + 0.5 * size
        dctr = reg[:, d:d + 1]
        dsize = jnp.minimum(reg[:, d + 2:d + 3], _CLIP)
        pctr = dctr * size + ctr
        psize = jnp.exp(dsize) * size
        lows.append(jnp.clip(pctr - 0.5 * psize, 0.0, _IM))
        highs.append(jnp.clip(pctr + 0.5 * psize, 0.0, _IM))
    l0, l1 = lows
    h0, h1 = highs
    validf = ((h0 > l0) & (h1 > l1)).astype(jnp.float32)
    s = jnp.where(validf > 0.0, scores, -1.0)

    tri = (
        jax.lax.broadcasted_iota(jnp.int32, (_PRE, _PRE), 0)
        >= jax.lax.broadcasted_iota(jnp.int32, (_PRE, _PRE), 1)
    ).astype(jnp.float32)
    invf = 1.0 - validf
    csV = jnp.dot(tri, validf, preferred_element_type=jnp.float32)
    csI = jnp.dot(tri, invf, preferred_element_type=jnp.float32)
    nV = jnp.sum(validf)
    pos = validf * (csV - 1.0) + invf * (nV + csI - 1.0)
    pos_i = pos.astype(jnp.int32)
    cols_n = jax.lax.broadcasted_iota(jnp.int32, (_PRE, _PRE), 1)
    perm = (pos_i == cols_n).astype(jnp.float32)

    M = jnp.concatenate([l0, l1, h0, h1, s, validf], axis=1)
    Ms = _dotT(perm, M)
    MsT = _dotT(M, perm)
    sl0, sl1 = Ms[:, 0:1], Ms[:, 1:2]
    sh0, sh1 = Ms[:, 2:3], Ms[:, 3:4]
    ss = Ms[:, 4:5]
    rl0, rl1 = MsT[0:1, :], MsT[1:2, :]
    rh0, rh1 = MsT[2:3, :], MsT[3:4, :]
    sval_r = MsT[5:6, :]

    area_c = jnp.maximum(sh0 - sl0, 0.0) * jnp.maximum(sh1 - sl1, 0.0)
    area_r = jnp.maximum(rh0 - rl0, 0.0) * jnp.maximum(rh1 - rl1, 0.0)
    w0 = jnp.maximum(jnp.minimum(sh0, rh0) - jnp.maximum(sl0, rl0), 0.0)
    w1 = jnp.maximum(jnp.minimum(sh1, rh1) - jnp.maximum(sl1, rl1), 0.0)
    inter = w0 * w1
    union = area_c + area_r - inter
    iou = inter / jnp.maximum(union, 1e-9)

    cols_r = jax.lax.broadcasted_iota(jnp.int32, (1, _PRE), 1)
    iou_s[...] = iou

    def body(i, sup):
        rowi = iou_s[pl.ds(i, 1), :]
        sup_i = jnp.sum(jnp.where(cols_r == i, sup, 0.0))
        cand = ((rowi > _NMS_THRESH) & (cols_r > i)).astype(jnp.float32)
        return jnp.maximum(sup, (1.0 - sup_i) * cand)

    sup_r = jax.lax.fori_loop(
        0, _PRE, body, jnp.zeros((1, _PRE), jnp.float32)
    )
    keep_r = 1.0 - sup_r

    A_r = keep_r * sval_r
    B_r = keep_r * (1.0 - sval_r)
    C_r = sup_r
    triU = (
        jax.lax.broadcasted_iota(jnp.int32, (_PRE, _PRE), 0)
        <= jax.lax.broadcasted_iota(jnp.int32, (_PRE, _PRE), 1)
    ).astype(jnp.float32)
    csA = jnp.dot(A_r, triU, preferred_element_type=jnp.float32)
    csB = jnp.dot(B_r, triU, preferred_element_type=jnp.float32)
    csC = jnp.dot(C_r, triU, preferred_element_type=jnp.float32)
    nA = jnp.sum(A_r)
    nB = jnp.sum(B_r)
    pos2_r = (
        A_r * (csA - 1.0)
        + B_r * (nA + csB - 1.0)
        + C_r * (nA + nB + csC - 1.0)
    ).astype(jnp.int32)
    rows_p = jax.lax.broadcasted_iota(jnp.int32, (_POST, _PRE), 0)
    sel = (pos2_r == rows_p).astype(jnp.float32)

    ident = (
        jax.lax.broadcasted_iota(jnp.int32, (_PRE, _PRE), 0)
        == jax.lax.broadcasted_iota(jnp.int32, (_PRE, _PRE), 1)
    ).astype(jnp.float32)
    keep_c = jax.lax.dot_general(
        ident, keep_r, (((1,), (1,)), ((), ())),
        preferred_element_type=jnp.float32,
    )
    masked = jnp.where(keep_c > 0.0, ss, _NEG)
    M2 = jnp.concatenate([sl0, sl1, sh0, sh1, masked], axis=1)
    out_ref[0] = jnp.dot(sel, M2, preferred_element_type=jnp.float32)


def kernel(anchors, objectness, box_regression):
    n, a, h, w = objectness.shape
    flat_obj = jnp.transpose(objectness, (0, 2, 3, 1)).reshape(n, -1)
    flat_reg = jnp.transpose(
        box_regression.reshape(n, a, 4, h, w), (0, 3, 4, 1, 2)
    ).reshape(n, -1, 4)
    logits, idx = jax.lax.top_k(flat_obj, _PRE)
    data = jnp.concatenate([anchors, flat_reg], axis=-1)
    idx3 = idx[..., None].astype(jnp.int32)
    obj3 = logits[..., None]
    num = a * h * w
    return pl.pallas_call(
        _rpn_kernel,
        grid=(n,),
        in_specs=[
            pl.BlockSpec((1, num, 8), lambda i: (i, 0, 0)),
            pl.BlockSpec((1, _PRE, 1), lambda i: (i, 0, 0)),
            pl.BlockSpec((1, _PRE, 1), lambda i: (i, 0, 0)),
        ],
        out_specs=pl.BlockSpec((1, _POST, 5), lambda i: (i, 0, 0)),
        out_shape=jax.ShapeDtypeStruct((n, _POST, 5), jnp.float32),
        scratch_shapes=[
            pltpu.VMEM((_PRE, _PRE), jnp.float32),
        ],
    )(data, idx3, obj3)


# speedup vs baseline: 6.4298x; 1.0029x over previous
import jax
import jax.numpy as jnp
import numpy as np
from jax.experimental import pallas as pl
from jax.experimental.pallas import tpu as pltpu

_PRE = 1000
_POST = 300
_NMS_THRESH = 0.7
_IM = 800.0
_CLIP = float(np.log(1000.0 / 16.0))
_NEG = -1e9
_CHUNK = 2400


def _dotT(a, b):
    return jax.lax.dot_general(
        a, b, (((0,), (0,)), ((), ())), preferred_element_type=jnp.float32
    )


def _rpn_kernel(data_ref, idx_ref, obj_ref, out_ref, iou_s):
    data = data_ref[0]
    idx = idx_ref[0]
    logits = obj_ref[0]

    num = data.shape[0]
    gathered = jnp.zeros((_PRE, 8), jnp.float32)
    for c in range(0, num, _CHUNK):
        blk = data[c:c + _CHUNK, :]
        cols = c + jax.lax.broadcasted_iota(jnp.int32, (_PRE, _CHUNK), 1)
        oh = (idx == cols).astype(jnp.float32)
        gathered = gathered + jnp.dot(
            oh, blk, preferred_element_type=jnp.float32
        )

    anc = gathered[:, 0:4]
    reg = gathered[:, 4:8]
    scores = jax.nn.sigmoid(logits)

    lows, highs = [], []
    for d in range(2):
        size = anc[:, d + 2:d + 3] - anc[:, d:d + 1]
        ctr = anc[:, d:d + 1] + 0.5 * size
        dctr = reg[:, d:d + 1]
        dsize = jnp.minimum(reg[:, d + 2:d + 3], _CLIP)
        pctr = dctr * size + ctr
        psize = jnp.exp(dsize) * size
        lows.append(jnp.clip(pctr - 0.5 * psize, 0.0, _IM))
        highs.append(jnp.clip(pctr + 0.5 * psize, 0.0, _IM))
    l0, l1 = lows
    h0, h1 = highs
    validf = ((h0 > l0) & (h1 > l1)).astype(jnp.float32)
    s = jnp.where(validf > 0.0, scores, -1.0)

    tri = (
        jax.lax.broadcasted_iota(jnp.int32, (_PRE, _PRE), 0)
        >= jax.lax.broadcasted_iota(jnp.int32, (_PRE, _PRE), 1)
    ).astype(jnp.float32)
    invf = 1.0 - validf
    csV = jnp.dot(tri, validf, preferred_element_type=jnp.float32)
    csI = jnp.dot(tri, invf, preferred_element_type=jnp.float32)
    nV = jnp.sum(validf)
    pos = validf * (csV - 1.0) + invf * (nV + csI - 1.0)
    pos_i = pos.astype(jnp.int32)
    cols_n = jax.lax.broadcasted_iota(jnp.int32, (_PRE, _PRE), 1)
    perm = (pos_i == cols_n).astype(jnp.float32)

    M = jnp.concatenate([l0, l1, h0, h1, s, validf], axis=1)
    Ms = _dotT(perm, M)
    MsT = _dotT(M, perm)
    sl0, sl1 = Ms[:, 0:1], Ms[:, 1:2]
    sh0, sh1 = Ms[:, 2:3], Ms[:, 3:4]
    ss = Ms[:, 4:5]
    rl0, rl1 = MsT[0:1, :], MsT[1:2, :]
    rh0, rh1 = MsT[2:3, :], MsT[3:4, :]
    sval_r = MsT[5:6, :]

    area_c = jnp.maximum(sh0 - sl0, 0.0) * jnp.maximum(sh1 - sl1, 0.0)
    area_r = jnp.maximum(rh0 - rl0, 0.0) * jnp.maximum(rh1 - rl1, 0.0)
    w0 = jnp.maximum(jnp.minimum(sh0, rh0) - jnp.maximum(sl0, rl0), 0.0)
    w1 = jnp.maximum(jnp.minimum(sh1, rh1) - jnp.maximum(sl1, rl1), 0.0)
    inter = w0 * w1
    union = area_c + area_r - inter
    iou = inter / jnp.maximum(union, 1e-9)

    cols_r = jax.lax.broadcasted_iota(jnp.int32, (1, _PRE), 1)
    rows_n = jax.lax.broadcasted_iota(jnp.int32, (_PRE, _PRE), 0)
    iou_s[...] = ((iou > _NMS_THRESH) & (cols_n > rows_n)).astype(jnp.float32)

    def body(i, sup):
        rowi = iou_s[pl.ds(i, 1), :]
        sup_i = jnp.sum(jnp.where(cols_r == i, sup, 0.0))
        return jnp.maximum(sup, (1.0 - sup_i) * rowi)

    sup_r = jax.lax.fori_loop(
        0, _PRE, body, jnp.zeros((1, _PRE), jnp.float32)
    )
    keep_r = 1.0 - sup_r

    A_r = keep_r * sval_r
    B_r = keep_r * (1.0 - sval_r)
    C_r = sup_r
    triU = (
        jax.lax.broadcasted_iota(jnp.int32, (_PRE, _PRE), 0)
        <= jax.lax.broadcasted_iota(jnp.int32, (_PRE, _PRE), 1)
    ).astype(jnp.float32)
    csA = jnp.dot(A_r, triU, preferred_element_type=jnp.float32)
    csB = jnp.dot(B_r, triU, preferred_element_type=jnp.float32)
    csC = jnp.dot(C_r, triU, preferred_element_type=jnp.float32)
    nA = jnp.sum(A_r)
    nB = jnp.sum(B_r)
    pos2_r = (
        A_r * (csA - 1.0)
        + B_r * (nA + csB - 1.0)
        + C_r * (nA + nB + csC - 1.0)
    ).astype(jnp.int32)
    rows_p = jax.lax.broadcasted_iota(jnp.int32, (_POST, _PRE), 0)
    sel = (pos2_r == rows_p).astype(jnp.float32)

    ident = (
        jax.lax.broadcasted_iota(jnp.int32, (_PRE, _PRE), 0)
        == jax.lax.broadcasted_iota(jnp.int32, (_PRE, _PRE), 1)
    ).astype(jnp.float32)
    keep_c = jax.lax.dot_general(
        ident, keep_r, (((1,), (1,)), ((), ())),
        preferred_element_type=jnp.float32,
    )
    masked = jnp.where(keep_c > 0.0, ss, _NEG)
    M2 = jnp.concatenate([sl0, sl1, sh0, sh1, masked], axis=1)
    out_ref[0] = jnp.dot(sel, M2, preferred_element_type=jnp.float32)


def kernel(anchors, objectness, box_regression):
    n, a, h, w = objectness.shape
    flat_obj = jnp.transpose(objectness, (0, 2, 3, 1)).reshape(n, -1)
    flat_reg = jnp.transpose(
        box_regression.reshape(n, a, 4, h, w), (0, 3, 4, 1, 2)
    ).reshape(n, -1, 4)
    logits, idx = jax.lax.top_k(flat_obj, _PRE)
    data = jnp.concatenate([anchors, flat_reg], axis=-1)
    idx3 = idx[..., None].astype(jnp.int32)
    obj3 = logits[..., None]
    num = a * h * w
    return pl.pallas_call(
        _rpn_kernel,
        grid=(n,),
        in_specs=[
            pl.BlockSpec((1, num, 8), lambda i: (i, 0, 0)),
            pl.BlockSpec((1, _PRE, 1), lambda i: (i, 0, 0)),
            pl.BlockSpec((1, _PRE, 1), lambda i: (i, 0, 0)),
        ],
        out_specs=pl.BlockSpec((1, _POST, 5), lambda i: (i, 0, 0)),
        out_shape=jax.ShapeDtypeStruct((n, _POST, 5), jnp.float32),
        scratch_shapes=[
            pltpu.VMEM((_PRE, _PRE), jnp.float32),
        ],
    )(data, idx3, obj3)
